# Initial kernel scaffold; baseline (speedup 1.0000x reference)
import jax
import jax.numpy as jnp
from jax.experimental import pallas as pl


def kernel(x, edge_index, Wg, W1, b1, W2, b2):
    raise NotImplementedError("write your pallas kernel here")


# speedup vs baseline: 1.9838x; 1.9838x over previous
import functools

import jax
import jax.numpy as jnp
from jax import lax
from jax.experimental import pallas as pl
from jax.experimental.pallas import tpu as pltpu
from jax.experimental.pallas import tpu_sc as plsc

N_NODES = 10000
D = 256
DQ = 64
N_EXP = 8
N_PAD = 10240
TILES = 16
CH = 128
E_EDGES = 160000
SLAB = 4096
E_PAD = 163840
NSLAB = E_PAD // SLAB
NCHD = E_PAD // (TILES * CH)
BROWS = N_PAD // TILES
ACC_R = BROWS + 8
CAP = 12800
NCHMAX = CAP // CH
BLK = 256
N_BLKS = N_PAD // BLK
INV_BROWS = 1.0 / float(BROWS)


def _fill_2d(buf, rows, val16):
    def fr(i, carry):
        def fc(j, carry2):
            buf[i, pl.ds(j * 16, 16)] = val16
            return carry2
        return lax.fori_loop(0, buf.shape[1] // 16, fc, carry)
    lax.fori_loop(0, rows, fr, 0)


def _fill_1d(buf, val16):
    def fr(i, carry):
        buf[pl.ds(i * 16, 16)] = val16
        return carry
    lax.fori_loop(0, buf.shape[0] // 16, fr, 0)


def _sc_part_body(src_e, dst_e, src_out, dst_out, cnt_out,
                  slab_s, slab_d, src_l, dst_l, cnt_v, sem):
    c = lax.axis_index("c")
    s = lax.axis_index("s")

    @pl.when(c == 0)
    def _():
        _fill_1d(src_l, jnp.zeros((16,), jnp.int32))
        _fill_1d(dst_l, jnp.full((16,), BROWS, jnp.int32))
        lane = lax.iota(jnp.int32, 16)
        base_row = s * BROWS

        def do_slab(k, cur):
            pltpu.sync_copy(src_e.at[k], slab_s)
            pltpu.sync_copy(dst_e.at[k], slab_d)

            def group(g, cur2):
                d = slab_d[pl.ds(g * 16, 16)]
                v = slab_s[pl.ds(g * 16, 16)]
                b = (d.astype(jnp.float32)
                     * jnp.float32(INV_BROWS)).astype(jnp.int32)
                gidx = k * SLAB + g * 16 + lane
                m = (b == s) & (gidx < E_EDGES)
                dl = d - base_row

                @pl.when(cur2 <= CAP - 16)
                def _():
                    plsc.store_compressed(src_l.at[pl.ds(cur2, 16)], v, mask=m)
                    plsc.store_compressed(dst_l.at[pl.ds(cur2, 16)], dl, mask=m)
                pc = plsc.all_reduce_population_count(m)
                return cur2 + jnp.max(pc, axis=0)
            return lax.fori_loop(0, SLAB // 16, group, cur)
        cur = lax.fori_loop(0, NSLAB, do_slab, jnp.int32(0))
        cur = jnp.minimum(cur, CAP)
        cnt_v[...] = jnp.full((16,), 0, jnp.int32) + cur
        pltpu.sync_copy(src_l, src_out.at[s])
        pltpu.sync_copy(dst_l, dst_out.at[s])
        pltpu.sync_copy(cnt_v, cnt_out.at[s])


@functools.cache
def _get_sc_part():
    return pl.kernel(
        _sc_part_body,
        out_type=(jax.ShapeDtypeStruct((TILES, CAP), jnp.int32),
                  jax.ShapeDtypeStruct((TILES, CAP), jnp.int32),
                  jax.ShapeDtypeStruct((TILES, 16), jnp.int32)),
        mesh=plsc.VectorSubcoreMesh(core_axis_name="c", subcore_axis_name="s"),
        scratch_types=[
            pltpu.VMEM((SLAB,), jnp.int32),
            pltpu.VMEM((SLAB,), jnp.int32),
            pltpu.VMEM((CAP,), jnp.int32),
            pltpu.VMEM((CAP,), jnp.int32),
            pltpu.VMEM((16,), jnp.int32),
            pltpu.SemaphoreType.DMA,
        ],
        compiler_params=pltpu.CompilerParams(use_tc_tiling_on_sc=False,
                                             needs_layout_passes=False),
    )


def _sc_agg_body(h_q0, h_q1, h_q2, h_q3, src_l3, dst_l3, cnts,
                 a_q0, a_q1, a_q2, a_q3,
                 src_v, dst_v, cnt_v, rows_a, rows_b, acc, sem_a, sem_b):
    c = lax.axis_index("c")
    s = lax.axis_index("s")
    pltpu.sync_copy(src_l3.at[s], src_v)
    pltpu.sync_copy(dst_l3.at[s], dst_v)
    pltpu.sync_copy(cnts.at[s], cnt_v)
    cnt = jnp.max(cnt_v[...], axis=0)
    nch = (cnt + (CH - 1)) // CH

    def one_pass(h_ref, out_ref):
        _fill_2d(acc, ACC_R, jnp.zeros((16,), jnp.float32))

        def add_chunk(ci, rows):
            def add_group(g, carry2):
                dlv = dst_v[ci, pl.ds(g * 16, 16)]
                for j in range(16):
                    dl = dlv[j]
                    i = g * 16 + j
                    for k4 in range(DQ // 16):
                        sl = pl.ds(k4 * 16, 16)
                        acc[dl, sl] = acc[dl, sl] + rows[i, sl]
                return carry2
            lax.fori_loop(0, CH // 16, add_group, 0)

        @pl.when(nch > 0)
        def _():
            pltpu.async_copy(h_ref.at[src_v.at[0]], rows_a, sem_a)

        def body(ci, carry):
            nxt = ci + 1

            @pl.when(nxt < nch)
            def _():
                @pl.when(nxt % 2 == 0)
                def _():
                    pltpu.async_copy(h_ref.at[src_v.at[nxt]], rows_a, sem_a)

                @pl.when(nxt % 2 == 1)
                def _():
                    pltpu.async_copy(h_ref.at[src_v.at[nxt]], rows_b, sem_b)

            @pl.when(ci % 2 == 0)
            def _():
                pltpu.make_async_copy(h_ref.at[pl.ds(0, CH)], rows_a,
                                      sem_a).wait()
                add_chunk(ci, rows_a)

            @pl.when(ci % 2 == 1)
            def _():
                pltpu.make_async_copy(h_ref.at[pl.ds(0, CH)], rows_b,
                                      sem_b).wait()
                add_chunk(ci, rows_b)
            return carry
        lax.fori_loop(0, nch, body, 0)
        pltpu.sync_copy(acc.at[pl.ds(0, BROWS)],
                        out_ref.at[pl.ds(s * BROWS, BROWS)])

    @pl.when(c == 0)
    def _():
        one_pass(h_q0, a_q0)
        one_pass(h_q1, a_q1)

    @pl.when(c == 1)
    def _():
        one_pass(h_q2, a_q2)
        one_pass(h_q3, a_q3)


@functools.cache
def _get_sc_agg():
    return pl.kernel(
        _sc_agg_body,
        out_type=tuple(jax.ShapeDtypeStruct((N_PAD, DQ), jnp.float32)
                       for _ in range(4)),
        mesh=plsc.VectorSubcoreMesh(core_axis_name="c", subcore_axis_name="s"),
        scratch_types=[
            pltpu.VMEM((NCHMAX, CH), jnp.int32),
            pltpu.VMEM((NCHMAX, CH), jnp.int32),
            pltpu.VMEM((16,), jnp.int32),
            pltpu.VMEM((CH, DQ), jnp.float32),
            pltpu.VMEM((CH, DQ), jnp.float32),
            pltpu.VMEM((ACC_R, DQ), jnp.float32),
            pltpu.SemaphoreType.DMA,
            pltpu.SemaphoreType.DMA,
        ],
        compiler_params=pltpu.CompilerParams(use_tc_tiling_on_sc=False,
                                             needs_layout_passes=False),
    )


def _sc_deg_body(dst_idx, deg_out, idxd_v, ones_v, zrows, acc, sem):
    c = lax.axis_index("c")
    s = lax.axis_index("s")

    @pl.when(c == 0)
    def _():
        _fill_2d(ones_v, CH, jnp.ones((16,), jnp.float32))
        _fill_2d(zrows, CH, jnp.zeros((16,), jnp.float32))
        for r in range(BROWS // CH):
            pltpu.sync_copy(zrows, acc.at[pl.ds(s * BROWS + r * CH, CH)])
        pltpu.sync_copy(dst_idx.at[s], idxd_v)
        plsc.subcore_barrier()

        def body(ci, carry):
            pltpu.sync_copy(ones_v, acc.at[idxd_v.at[ci]], add=True)
            return carry
        lax.fori_loop(0, NCHD, body, 0)
        plsc.subcore_barrier()
        pltpu.sync_copy(acc.at[pl.ds(s * BROWS, BROWS)],
                        deg_out.at[pl.ds(s * BROWS, BROWS)])


@functools.cache
def _get_sc_deg():
    return pl.kernel(
        _sc_deg_body,
        out_type=jax.ShapeDtypeStruct((N_PAD, DQ), jnp.float32),
        mesh=plsc.VectorSubcoreMesh(core_axis_name="c", subcore_axis_name="s"),
        scratch_types=[
            pltpu.VMEM((NCHD, CH), jnp.int32),
            pltpu.VMEM((CH, DQ), jnp.float32),
            pltpu.VMEM((CH, DQ), jnp.float32),
            pltpu.VMEM_SHARED((N_PAD, DQ), jnp.float32),
            pltpu.SemaphoreType.DMA,
        ],
        compiler_params=pltpu.CompilerParams(use_tc_tiling_on_sc=False,
                                             needs_layout_passes=False),
    )


def _tc_moe_body(last, h_q0, h_q1, h_q2, h_q3, a_q0, a_q1, a_q2, a_q3, degr,
                 wg, w1, b1, w2, b2, o_q0, o_q1, o_q2, o_q3):
    inv = 1.0 / jnp.maximum(degr[...], 1.0)
    hq = [h[...] + a[...] * inv
          for h, a in ((h_q0, a_q0), (h_q1, a_q1), (h_q2, a_q2),
                       (h_q3, a_q3))]
    hn = jnp.concatenate(hq, axis=1)

    hn_b = hn.astype(jnp.bfloat16)
    logits = jnp.dot(hn_b, wg[...].astype(jnp.bfloat16),
                     preferred_element_type=jnp.float32)
    lane = lax.broadcasted_iota(jnp.int32, (BLK, 128), 1)
    logits = jnp.where(lane < N_EXP, logits, jnp.float32(-1e30))
    m = jnp.max(logits, axis=1, keepdims=True)
    p = jnp.exp(logits - m)
    p = jnp.where(lane < N_EXP, p, 0.0)
    z = jnp.sum(p, axis=1, keepdims=True)
    probs = p / z

    m1 = jnp.max(probs, axis=1, keepdims=True)
    i1 = jnp.min(jnp.where(probs == m1, lane, 128), axis=1, keepdims=True)
    p2 = jnp.where(lane == i1, -1.0, probs)
    m2 = jnp.max(p2, axis=1, keepdims=True)
    i2 = jnp.min(jnp.where(p2 == m2, lane, 128), axis=1, keepdims=True)
    den = m1 + m2 + 1e-9
    g1 = m1 / den
    g2 = m2 / den

    acc = jnp.zeros((BLK, D), jnp.float32)
    for e in range(N_EXP):
        w = jnp.where(i1 == e, g1, 0.0) + jnp.where(i2 == e, g2, 0.0)
        hid = jnp.maximum(
            jnp.dot(hn_b, w1[e].astype(jnp.bfloat16),
                    preferred_element_type=jnp.float32)
            + b1[e][None, :], 0.0)
        oute = (jnp.dot(hid.astype(jnp.bfloat16), w2[e].astype(jnp.bfloat16),
                        preferred_element_type=jnp.float32)
                + b2[e][None, :])
        w_r = w.astype(jnp.bfloat16).astype(jnp.float32)
        oute_r = oute.astype(jnp.bfloat16).astype(jnp.float32)
        acc = acc + w_r * oute_r
    if not last:
        acc = jnp.maximum(acc, 0.0)
    o_q0[...] = acc[:, 0 * DQ:1 * DQ]
    o_q1[...] = acc[:, 1 * DQ:2 * DQ]
    o_q2[...] = acc[:, 2 * DQ:3 * DQ]
    o_q3[...] = acc[:, 3 * DQ:4 * DQ]


def _tc_moe(last, hqs, aqs, deg, wg, w1, b1, w2, b2):
    bs_q = pl.BlockSpec((BLK, DQ), lambda i: (i, 0))
    bs_wg = pl.BlockSpec((D, 128), lambda i: (0, 0))
    bs_w = pl.BlockSpec((N_EXP, D, D), lambda i: (0, 0, 0))
    bs_b = pl.BlockSpec((N_EXP, D), lambda i: (0, 0))
    return pl.pallas_call(
        functools.partial(_tc_moe_body, last),
        grid=(N_BLKS,),
        in_specs=[bs_q] * 9 + [bs_wg, bs_w, bs_b, bs_w, bs_b],
        out_specs=[bs_q] * 4,
        out_shape=tuple(jax.ShapeDtypeStruct((N_PAD, DQ), jnp.float32)
                        for _ in range(4)),
    )(*hqs, *aqs, deg, wg, w1, b1, w2, b2)


def kernel(x, edge_index, Wg, W1, b1, W2, b2):
    n = x.shape[0]
    e = edge_index.shape[1]
    n_layers = Wg.shape[0]
    src = edge_index[0]
    dst = edge_index[1]
    src_es = jnp.concatenate(
        [src, jnp.zeros((E_PAD - e,), jnp.int32)]).reshape(NSLAB, SLAB)
    dst_es = jnp.concatenate(
        [dst, jnp.full((E_PAD - e,), n, jnp.int32)]).reshape(NSLAB, SLAB)
    dst_p = dst_es.reshape(TILES, NCHD, CH)
    xp = jnp.pad(x, ((0, N_PAD - n), (0, 0)))
    hqs = tuple(xp[:, q * DQ:(q + 1) * DQ] for q in range(4))
    wg_p = jnp.pad(Wg, ((0, 0), (0, 0), (0, 128 - N_EXP)))

    src_l, dst_l, cnts = _get_sc_part()(src_es, dst_es)
    src_l3 = src_l.reshape(TILES, NCHMAX, CH)
    dst_l3 = dst_l.reshape(TILES, NCHMAX, CH)
    deg = _get_sc_deg()(dst_p)
    for l in range(n_layers):
        aqs = _get_sc_agg()(*hqs, src_l3, dst_l3, cnts)
        hqs = _tc_moe(l == n_layers - 1, hqs, aqs, deg, wg_p[l], W1[l],
                      b1[l], W2[l], b2[l])
    return jnp.concatenate(hqs, axis=1)[:n]


# speedup vs baseline: 2.5298x; 1.2752x over previous
import functools

import jax
import jax.numpy as jnp
from jax import lax
from jax.experimental import pallas as pl
from jax.experimental.pallas import tpu as pltpu
from jax.experimental.pallas import tpu_sc as plsc

N_NODES = 10000
D = 256
DQ = 64
N_EXP = 8
N_PAD = 10240
TILES = 16
NB = 32
CH = 64
E_EDGES = 160000
SLAB = 4096
E_PAD = 163840
NSLAB = E_PAD // SLAB
NCHD = E_PAD // (TILES * 128)
BROWS = N_PAD // NB
ACC_R = BROWS + 4
CAP = 6144
NCHMAX = CAP // CH
BLK = 256
N_BLKS = N_PAD // BLK
INV_BROWS = 1.0 / float(BROWS)


def _fill_2d(buf, rows, val16):
    def fr(i, carry):
        def fc(j, carry2):
            buf[i, pl.ds(j * 16, 16)] = val16
            return carry2
        return lax.fori_loop(0, buf.shape[1] // 16, fc, carry)
    lax.fori_loop(0, rows, fr, 0)


def _fill_1d(buf, val16):
    def fr(i, carry):
        buf[pl.ds(i * 16, 16)] = val16
        return carry
    lax.fori_loop(0, buf.shape[0] // 16, fr, 0)


def _sc_part_body(src_e, dst_e, src_out, dst_out, cnt_out,
                  slab_s, slab_d, src_l, dst_l, cnt_v, sem):
    c = lax.axis_index("c")
    s = lax.axis_index("s")
    myb = c * TILES + s
    _fill_1d(src_l, jnp.zeros((16,), jnp.int32))
    _fill_1d(dst_l, jnp.full((16,), BROWS, jnp.int32))
    lane = lax.iota(jnp.int32, 16)
    base_row = myb * BROWS

    def do_slab(k, cur):
        pltpu.sync_copy(src_e.at[k], slab_s)
        pltpu.sync_copy(dst_e.at[k], slab_d)

        def group(g, cur2):
            d = slab_d[pl.ds(g * 16, 16)]
            v = slab_s[pl.ds(g * 16, 16)]
            b = (d.astype(jnp.float32)
                 * jnp.float32(INV_BROWS)).astype(jnp.int32)
            gidx = k * SLAB + g * 16 + lane
            m = (b == myb) & (gidx < E_EDGES)
            dl = d - base_row

            @pl.when(cur2 <= CAP - 16)
            def _():
                plsc.store_compressed(src_l.at[pl.ds(cur2, 16)], v, mask=m)
                plsc.store_compressed(dst_l.at[pl.ds(cur2, 16)], dl, mask=m)
            pc = plsc.all_reduce_population_count(m)
            return cur2 + jnp.max(pc, axis=0)
        return lax.fori_loop(0, SLAB // 16, group, cur)
    cur = lax.fori_loop(0, NSLAB, do_slab, jnp.int32(0))
    cur = jnp.minimum(cur, CAP)
    cnt_v[...] = jnp.full((16,), 0, jnp.int32) + cur
    pltpu.sync_copy(src_l, src_out.at[myb])
    pltpu.sync_copy(dst_l, dst_out.at[myb])
    pltpu.sync_copy(cnt_v, cnt_out.at[myb])


@functools.cache
def _get_sc_part():
    return pl.kernel(
        _sc_part_body,
        out_type=(jax.ShapeDtypeStruct((NB, CAP), jnp.int32),
                  jax.ShapeDtypeStruct((NB, CAP), jnp.int32),
                  jax.ShapeDtypeStruct((NB, 16), jnp.int32)),
        mesh=plsc.VectorSubcoreMesh(core_axis_name="c", subcore_axis_name="s"),
        scratch_types=[
            pltpu.VMEM((SLAB,), jnp.int32),
            pltpu.VMEM((SLAB,), jnp.int32),
            pltpu.VMEM((CAP,), jnp.int32),
            pltpu.VMEM((CAP,), jnp.int32),
            pltpu.VMEM((16,), jnp.int32),
            pltpu.SemaphoreType.DMA,
        ],
        compiler_params=pltpu.CompilerParams(use_tc_tiling_on_sc=False,
                                             needs_layout_passes=False),
    )


def _sc_agg_body(h, src_l3, dst_l3, cnts, a_out,
                 src_v, dst_v, cnt_v, rows_a, rows_b, acc, sem_a, sem_b):
    c = lax.axis_index("c")
    s = lax.axis_index("s")
    myb = c * TILES + s
    pltpu.sync_copy(src_l3.at[myb], src_v)
    pltpu.sync_copy(dst_l3.at[myb], dst_v)
    pltpu.sync_copy(cnts.at[myb], cnt_v)
    cnt = jnp.max(cnt_v[...], axis=0)
    nch = (cnt + (CH - 1)) // CH

    _fill_2d(acc, ACC_R, jnp.zeros((16,), jnp.float32))

    def add_chunk(ci, rows):
        def add_group(g, carry2):
            dlv = dst_v[ci, pl.ds(g * 16, 16)]
            for j in range(16):
                dl = dlv[j]
                i = g * 16 + j
                for k16 in range(D // 16):
                    sl = pl.ds(k16 * 16, 16)
                    plsc.addupdate(acc.at[dl, sl], rows[i, sl])
            return carry2
        lax.fori_loop(0, CH // 16, add_group, 0)

    @pl.when(nch > 0)
    def _():
        pltpu.async_copy(h.at[src_v.at[0]], rows_a, sem_a)

    def body(ci, carry):
        nxt = ci + 1

        @pl.when(nxt < nch)
        def _():
            @pl.when(nxt % 2 == 0)
            def _():
                pltpu.async_copy(h.at[src_v.at[nxt]], rows_a, sem_a)

            @pl.when(nxt % 2 == 1)
            def _():
                pltpu.async_copy(h.at[src_v.at[nxt]], rows_b, sem_b)

        @pl.when(ci % 2 == 0)
        def _():
            pltpu.make_async_copy(h.at[pl.ds(0, CH)], rows_a, sem_a).wait()
            add_chunk(ci, rows_a)

        @pl.when(ci % 2 == 1)
        def _():
            pltpu.make_async_copy(h.at[pl.ds(0, CH)], rows_b, sem_b).wait()
            add_chunk(ci, rows_b)
        return carry
    lax.fori_loop(0, nch, body, 0)
    pltpu.sync_copy(acc.at[pl.ds(0, BROWS)],
                    a_out.at[pl.ds(myb * BROWS, BROWS)])


@functools.cache
def _get_sc_agg():
    return pl.kernel(
        _sc_agg_body,
        out_type=jax.ShapeDtypeStruct((N_PAD, D), jnp.float32),
        mesh=plsc.VectorSubcoreMesh(core_axis_name="c", subcore_axis_name="s"),
        scratch_types=[
            pltpu.VMEM((NCHMAX, CH), jnp.int32),
            pltpu.VMEM((NCHMAX, CH), jnp.int32),
            pltpu.VMEM((16,), jnp.int32),
            pltpu.VMEM((CH, D), jnp.float32),
            pltpu.VMEM((CH, D), jnp.float32),
            pltpu.VMEM((ACC_R, D), jnp.float32),
            pltpu.SemaphoreType.DMA,
            pltpu.SemaphoreType.DMA,
        ],
        compiler_params=pltpu.CompilerParams(use_tc_tiling_on_sc=False,
                                             needs_layout_passes=False),
    )


def _sc_deg_body(dst_idx, deg_out, idxd_v, ones_v, zrows, acc, sem):
    c = lax.axis_index("c")
    s = lax.axis_index("s")

    @pl.when(c == 0)
    def _():
        _fill_2d(ones_v, 128, jnp.ones((16,), jnp.float32))
        _fill_2d(zrows, 128, jnp.zeros((16,), jnp.float32))
        for r in range(5):
            pltpu.sync_copy(zrows, acc.at[pl.ds(s * 640 + r * 128, 128)])
        pltpu.sync_copy(dst_idx.at[s], idxd_v)
        plsc.subcore_barrier()

        def body(ci, carry):
            pltpu.sync_copy(ones_v, acc.at[idxd_v.at[ci]], add=True)
            return carry
        lax.fori_loop(0, NCHD, body, 0)
        plsc.subcore_barrier()
        pltpu.sync_copy(acc.at[pl.ds(s * 640, 640)],
                        deg_out.at[pl.ds(s * 640, 640)])


@functools.cache
def _get_sc_deg():
    return pl.kernel(
        _sc_deg_body,
        out_type=jax.ShapeDtypeStruct((N_PAD, DQ), jnp.float32),
        mesh=plsc.VectorSubcoreMesh(core_axis_name="c", subcore_axis_name="s"),
        scratch_types=[
            pltpu.VMEM((NCHD, 128), jnp.int32),
            pltpu.VMEM((128, DQ), jnp.float32),
            pltpu.VMEM((128, DQ), jnp.float32),
            pltpu.VMEM_SHARED((N_PAD, DQ), jnp.float32),
            pltpu.SemaphoreType.DMA,
        ],
        compiler_params=pltpu.CompilerParams(use_tc_tiling_on_sc=False,
                                             needs_layout_passes=False),
    )


def _tc_moe_body(last, h_r, a_r, degr, wg, w1, b1, w2, b2, o_r):
    inv64 = 1.0 / jnp.maximum(degr[...], 1.0)
    inv = jnp.concatenate([inv64] * 4, axis=1)
    hn = h_r[...] + a_r[...] * inv

    hn_b = hn.astype(jnp.bfloat16)
    logits = jnp.dot(hn_b, wg[...].astype(jnp.bfloat16),
                     preferred_element_type=jnp.float32)
    lane = lax.broadcasted_iota(jnp.int32, (BLK, 128), 1)
    logits = jnp.where(lane < N_EXP, logits, jnp.float32(-1e30))
    m = jnp.max(logits, axis=1, keepdims=True)
    p = jnp.exp(logits - m)
    p = jnp.where(lane < N_EXP, p, 0.0)
    z = jnp.sum(p, axis=1, keepdims=True)
    probs = p / z

    m1 = jnp.max(probs, axis=1, keepdims=True)
    i1 = jnp.min(jnp.where(probs == m1, lane, 128), axis=1, keepdims=True)
    p2 = jnp.where(lane == i1, -1.0, probs)
    m2 = jnp.max(p2, axis=1, keepdims=True)
    i2 = jnp.min(jnp.where(p2 == m2, lane, 128), axis=1, keepdims=True)
    den = m1 + m2 + 1e-9
    g1 = m1 / den
    g2 = m2 / den

    acc = jnp.zeros((BLK, D), jnp.float32)
    for e in range(N_EXP):
        w = jnp.where(i1 == e, g1, 0.0) + jnp.where(i2 == e, g2, 0.0)
        hid = jnp.maximum(
            jnp.dot(hn_b, w1[e].astype(jnp.bfloat16),
                    preferred_element_type=jnp.float32)
            + b1[e][None, :], 0.0)
        oute = (jnp.dot(hid.astype(jnp.bfloat16), w2[e].astype(jnp.bfloat16),
                        preferred_element_type=jnp.float32)
                + b2[e][None, :])
        w_r = w.astype(jnp.bfloat16).astype(jnp.float32)
        oute_r = oute.astype(jnp.bfloat16).astype(jnp.float32)
        acc = acc + w_r * oute_r
    if not last:
        acc = jnp.maximum(acc, 0.0)
    o_r[...] = acc


def _tc_moe(last, h, a, deg, wg, w1, b1, w2, b2):
    bs_h = pl.BlockSpec((BLK, D), lambda i: (i, 0))
    bs_d = pl.BlockSpec((BLK, DQ), lambda i: (i, 0))
    bs_wg = pl.BlockSpec((D, 128), lambda i: (0, 0))
    bs_w = pl.BlockSpec((N_EXP, D, D), lambda i: (0, 0, 0))
    bs_b = pl.BlockSpec((N_EXP, D), lambda i: (0, 0))
    return pl.pallas_call(
        functools.partial(_tc_moe_body, last),
        grid=(N_BLKS,),
        in_specs=[bs_h, bs_h, bs_d, bs_wg, bs_w, bs_b, bs_w, bs_b],
        out_specs=bs_h,
        out_shape=jax.ShapeDtypeStruct((N_PAD, D), jnp.float32),
    )(h, a, deg, wg, w1, b1, w2, b2)


def kernel(x, edge_index, Wg, W1, b1, W2, b2):
    n = x.shape[0]
    e = edge_index.shape[1]
    n_layers = Wg.shape[0]
    src = edge_index[0]
    dst = edge_index[1]
    src_es = jnp.concatenate(
        [src, jnp.zeros((E_PAD - e,), jnp.int32)]).reshape(NSLAB, SLAB)
    dst_es = jnp.concatenate(
        [dst, jnp.full((E_PAD - e,), n, jnp.int32)]).reshape(NSLAB, SLAB)
    dst_p = dst_es.reshape(TILES, NCHD, 128)
    h = jnp.pad(x, ((0, N_PAD - n), (0, 0)))
    wg_p = jnp.pad(Wg, ((0, 0), (0, 0), (0, 128 - N_EXP)))

    src_l, dst_l, cnts = _get_sc_part()(src_es, dst_es)
    src_l3 = src_l.reshape(NB, NCHMAX, CH)
    dst_l3 = dst_l.reshape(NB, NCHMAX, CH)
    deg = _get_sc_deg()(dst_p)
    for l in range(n_layers):
        a = _get_sc_agg()(h, src_l3, dst_l3, cnts)
        h = _tc_moe(l == n_layers - 1, h, a, deg, wg_p[l], W1[l],
                    b1[l], W2[l], b2[l])
    return h[:n]


# speedup vs baseline: 3.9973x; 1.5801x over previous
import functools

import jax
import jax.numpy as jnp
from jax import lax
from jax.experimental import pallas as pl
from jax.experimental.pallas import tpu as pltpu
from jax.experimental.pallas import tpu_sc as plsc

N_NODES = 10000
D = 256
DQ = 64
N_EXP = 8
N_PAD = 10240
TILES = 16
NB = 32
CH = 64
E_EDGES = 160000
SLAB = 4096
E_PAD = 163840
NSLAB = E_PAD // SLAB
NCHD = E_PAD // (TILES * 128)
BROWS = N_PAD // NB
ACC_R = BROWS + 4
CAP = 6144
NCHMAX = CAP // CH
BLK = 256
N_BLKS = N_PAD // BLK
INV_BROWS = 1.0 / float(BROWS)


def _fill_2d(buf, rows, val16):
    def fr(i, carry):
        def fc(j, carry2):
            buf[i, pl.ds(j * 16, 16)] = val16
            return carry2
        return lax.fori_loop(0, buf.shape[1] // 16, fc, carry)
    lax.fori_loop(0, rows, fr, 0)


def _fill_1d(buf, val16):
    def fr(i, carry):
        buf[pl.ds(i * 16, 16)] = val16
        return carry
    lax.fori_loop(0, buf.shape[0] // 16, fr, 0)


def _sc_part_body(src_e, dst_e, src_out, dst_out, cnt_out,
                  slab_s, slab_d, src_l, dst_l, cnt_v, sem):
    c = lax.axis_index("c")
    s = lax.axis_index("s")
    myb = c * TILES + s
    _fill_1d(src_l, jnp.zeros((16,), jnp.int32))
    _fill_1d(dst_l, jnp.full((16,), BROWS, jnp.int32))
    lane = lax.iota(jnp.int32, 16)
    base_row = myb * BROWS

    def do_slab(k, cur):
        pltpu.sync_copy(src_e.at[k], slab_s)
        pltpu.sync_copy(dst_e.at[k], slab_d)

        def group(g, cur2):
            d = slab_d[pl.ds(g * 16, 16)]
            v = slab_s[pl.ds(g * 16, 16)]
            b = (d.astype(jnp.float32)
                 * jnp.float32(INV_BROWS)).astype(jnp.int32)
            gidx = k * SLAB + g * 16 + lane
            m = (b == myb) & (gidx < E_EDGES)
            dl = d - base_row

            @pl.when(cur2 <= CAP - 16)
            def _():
                plsc.store_compressed(src_l.at[pl.ds(cur2, 16)], v, mask=m)
                plsc.store_compressed(dst_l.at[pl.ds(cur2, 16)], dl, mask=m)
            pc = plsc.all_reduce_population_count(m)
            return cur2 + jnp.max(pc, axis=0)
        return lax.fori_loop(0, SLAB // 16, group, cur)
    cur = lax.fori_loop(0, NSLAB, do_slab, jnp.int32(0))
    cur = jnp.minimum(cur, CAP)
    cnt_v[...] = jnp.full((16,), 0, jnp.int32) + cur
    pltpu.sync_copy(src_l, src_out.at[myb])
    pltpu.sync_copy(dst_l, dst_out.at[myb])
    pltpu.sync_copy(cnt_v, cnt_out.at[myb])


@functools.cache
def _get_sc_part():
    return pl.kernel(
        _sc_part_body,
        out_type=(jax.ShapeDtypeStruct((NB, CAP), jnp.int32),
                  jax.ShapeDtypeStruct((NB, CAP), jnp.int32),
                  jax.ShapeDtypeStruct((NB, 16), jnp.int32)),
        mesh=plsc.VectorSubcoreMesh(core_axis_name="c", subcore_axis_name="s"),
        scratch_types=[
            pltpu.VMEM((SLAB,), jnp.int32),
            pltpu.VMEM((SLAB,), jnp.int32),
            pltpu.VMEM((CAP,), jnp.int32),
            pltpu.VMEM((CAP,), jnp.int32),
            pltpu.VMEM((16,), jnp.int32),
            pltpu.SemaphoreType.DMA,
        ],
        compiler_params=pltpu.CompilerParams(use_tc_tiling_on_sc=False,
                                             needs_layout_passes=False),
    )


def _sc_agg_body(h, src_l3, dst_l3, cnts, a_out,
                 src_v, dst_v, cnt_v, rows_a, rows_b, acc, sem_a, sem_b):
    c = lax.axis_index("c")
    s = lax.axis_index("s")
    myb = c * TILES + s
    pltpu.sync_copy(src_l3.at[myb], src_v)
    pltpu.sync_copy(dst_l3.at[myb], dst_v)
    pltpu.sync_copy(cnts.at[myb], cnt_v)
    cnt = jnp.max(cnt_v[...], axis=0)
    nch = (cnt + (CH - 1)) // CH

    _fill_2d(acc, ACC_R, jnp.zeros((16,), jnp.float32))

    def add_chunk(ci, rows):
        def add_group(g, carry2):
            dlv = dst_v[ci, pl.ds(g * 16, 16)]
            for j in range(16):
                dl = dlv[j]
                i = g * 16 + j
                xs = [rows[i, pl.ds(k16 * 16, 16)] for k16 in range(D // 16)]
                for k16 in range(D // 16):
                    plsc.addupdate(acc.at[dl, pl.ds(k16 * 16, 16)], xs[k16])
            return carry2
        lax.fori_loop(0, CH // 16, add_group, 0)

    @pl.when(nch > 0)
    def _():
        pltpu.async_copy(h.at[src_v.at[0]], rows_a, sem_a)

    def body(ci, carry):
        nxt = ci + 1

        @pl.when(nxt < nch)
        def _():
            @pl.when(nxt % 2 == 0)
            def _():
                pltpu.async_copy(h.at[src_v.at[nxt]], rows_a, sem_a)

            @pl.when(nxt % 2 == 1)
            def _():
                pltpu.async_copy(h.at[src_v.at[nxt]], rows_b, sem_b)

        @pl.when(ci % 2 == 0)
        def _():
            pltpu.make_async_copy(h.at[pl.ds(0, CH)], rows_a, sem_a).wait()
            add_chunk(ci, rows_a)

        @pl.when(ci % 2 == 1)
        def _():
            pltpu.make_async_copy(h.at[pl.ds(0, CH)], rows_b, sem_b).wait()
            add_chunk(ci, rows_b)
        return carry
    lax.fori_loop(0, nch, body, 0)
    pltpu.sync_copy(acc.at[pl.ds(0, BROWS)],
                    a_out.at[pl.ds(myb * BROWS, BROWS)])


@functools.cache
def _get_sc_agg():
    return pl.kernel(
        _sc_agg_body,
        out_type=jax.ShapeDtypeStruct((N_PAD, D), jnp.float32),
        mesh=plsc.VectorSubcoreMesh(core_axis_name="c", subcore_axis_name="s"),
        scratch_types=[
            pltpu.VMEM((NCHMAX, CH), jnp.int32),
            pltpu.VMEM((NCHMAX, CH), jnp.int32),
            pltpu.VMEM((16,), jnp.int32),
            pltpu.VMEM((CH, D), jnp.float32),
            pltpu.VMEM((CH, D), jnp.float32),
            pltpu.VMEM((ACC_R, D), jnp.float32),
            pltpu.SemaphoreType.DMA,
            pltpu.SemaphoreType.DMA,
        ],
        compiler_params=pltpu.CompilerParams(use_tc_tiling_on_sc=False,
                                             needs_layout_passes=False),
    )


def _sc_deg_body(dst_idx, deg_out, idxd_v, ones_v, zrows, acc, sem):
    c = lax.axis_index("c")
    s = lax.axis_index("s")

    @pl.when(c == 0)
    def _():
        _fill_2d(ones_v, 128, jnp.ones((16,), jnp.float32))
        _fill_2d(zrows, 128, jnp.zeros((16,), jnp.float32))
        for r in range(5):
            pltpu.sync_copy(zrows, acc.at[pl.ds(s * 640 + r * 128, 128)])
        pltpu.sync_copy(dst_idx.at[s], idxd_v)
        plsc.subcore_barrier()

        def body(ci, carry):
            pltpu.sync_copy(ones_v, acc.at[idxd_v.at[ci]], add=True)
            return carry
        lax.fori_loop(0, NCHD, body, 0)
        plsc.subcore_barrier()
        pltpu.sync_copy(acc.at[pl.ds(s * 640, 640)],
                        deg_out.at[pl.ds(s * 640, 640)])


@functools.cache
def _get_sc_deg():
    return pl.kernel(
        _sc_deg_body,
        out_type=jax.ShapeDtypeStruct((N_PAD, DQ), jnp.float32),
        mesh=plsc.VectorSubcoreMesh(core_axis_name="c", subcore_axis_name="s"),
        scratch_types=[
            pltpu.VMEM((NCHD, 128), jnp.int32),
            pltpu.VMEM((128, DQ), jnp.float32),
            pltpu.VMEM((128, DQ), jnp.float32),
            pltpu.VMEM_SHARED((N_PAD, DQ), jnp.float32),
            pltpu.SemaphoreType.DMA,
        ],
        compiler_params=pltpu.CompilerParams(use_tc_tiling_on_sc=False,
                                             needs_layout_passes=False),
    )


def _tc_moe_body(last, h_r, a_r, degr, wg, w1, b1, w2, b2, o_r):
    inv64 = 1.0 / jnp.maximum(degr[...], 1.0)
    inv = jnp.concatenate([inv64] * 4, axis=1)
    hn = h_r[...] + a_r[...] * inv

    hn_b = hn.astype(jnp.bfloat16)
    logits = jnp.dot(hn_b, wg[...].astype(jnp.bfloat16),
                     preferred_element_type=jnp.float32)
    lane = lax.broadcasted_iota(jnp.int32, (BLK, 128), 1)
    logits = jnp.where(lane < N_EXP, logits, jnp.float32(-1e30))
    m = jnp.max(logits, axis=1, keepdims=True)
    p = jnp.exp(logits - m)
    p = jnp.where(lane < N_EXP, p, 0.0)
    z = jnp.sum(p, axis=1, keepdims=True)
    probs = p / z

    m1 = jnp.max(probs, axis=1, keepdims=True)
    i1 = jnp.min(jnp.where(probs == m1, lane, 128), axis=1, keepdims=True)
    p2 = jnp.where(lane == i1, -1.0, probs)
    m2 = jnp.max(p2, axis=1, keepdims=True)
    i2 = jnp.min(jnp.where(p2 == m2, lane, 128), axis=1, keepdims=True)
    den = m1 + m2 + 1e-9
    g1 = m1 / den
    g2 = m2 / den

    acc = jnp.zeros((BLK, D), jnp.float32)
    for e in range(N_EXP):
        w = jnp.where(i1 == e, g1, 0.0) + jnp.where(i2 == e, g2, 0.0)
        hid = jnp.maximum(
            jnp.dot(hn_b, w1[e].astype(jnp.bfloat16),
                    preferred_element_type=jnp.float32)
            + b1[e][None, :], 0.0)
        oute = (jnp.dot(hid.astype(jnp.bfloat16), w2[e].astype(jnp.bfloat16),
                        preferred_element_type=jnp.float32)
                + b2[e][None, :])
        w_r = w.astype(jnp.bfloat16).astype(jnp.float32)
        oute_r = oute.astype(jnp.bfloat16).astype(jnp.float32)
        acc = acc + w_r * oute_r
    if not last:
        acc = jnp.maximum(acc, 0.0)
    o_r[...] = acc


def _tc_moe(last, h, a, deg, wg, w1, b1, w2, b2):
    bs_h = pl.BlockSpec((BLK, D), lambda i: (i, 0))
    bs_d = pl.BlockSpec((BLK, DQ), lambda i: (i, 0))
    bs_wg = pl.BlockSpec((D, 128), lambda i: (0, 0))
    bs_w = pl.BlockSpec((N_EXP, D, D), lambda i: (0, 0, 0))
    bs_b = pl.BlockSpec((N_EXP, D), lambda i: (0, 0))
    return pl.pallas_call(
        functools.partial(_tc_moe_body, last),
        grid=(N_BLKS,),
        in_specs=[bs_h, bs_h, bs_d, bs_wg, bs_w, bs_b, bs_w, bs_b],
        out_specs=bs_h,
        out_shape=jax.ShapeDtypeStruct((N_PAD, D), jnp.float32),
    )(h, a, deg, wg, w1, b1, w2, b2)


def kernel(x, edge_index, Wg, W1, b1, W2, b2):
    n = x.shape[0]
    e = edge_index.shape[1]
    n_layers = Wg.shape[0]
    src = edge_index[0]
    dst = edge_index[1]
    src_es = jnp.concatenate(
        [src, jnp.zeros((E_PAD - e,), jnp.int32)]).reshape(NSLAB, SLAB)
    dst_es = jnp.concatenate(
        [dst, jnp.full((E_PAD - e,), n, jnp.int32)]).reshape(NSLAB, SLAB)
    dst_p = dst_es.reshape(TILES, NCHD, 128)
    h = jnp.pad(x, ((0, N_PAD - n), (0, 0)))
    wg_p = jnp.pad(Wg, ((0, 0), (0, 0), (0, 128 - N_EXP)))

    src_l, dst_l, cnts = _get_sc_part()(src_es, dst_es)
    src_l3 = src_l.reshape(NB, NCHMAX, CH)
    dst_l3 = dst_l.reshape(NB, NCHMAX, CH)
    deg = _get_sc_deg()(dst_p)
    for l in range(n_layers):
        a = _get_sc_agg()(h, src_l3, dst_l3, cnts)
        h = _tc_moe(l == n_layers - 1, h, a, deg, wg_p[l], W1[l],
                    b1[l], W2[l], b2[l])
    return h[:n]
